# baseline (speedup 1.0000x reference)
import jax
import jax.numpy as jnp
from jax import lax
from jax.experimental import pallas as pl
from jax.experimental.pallas import tpu as pltpu
from jax.experimental.pallas import tpu_sc as plsc

NUM_FIELDS = 26
VOCAB = 100000
EMBED_DIM = 16
BATCH = 16384
NUM_PAIRS = (NUM_FIELDS * (NUM_FIELDS - 1)) // 2

_INFO = plsc.get_sparse_core_info()
NC = _INFO.num_cores
NS = _INFO.num_subcores
NW = NC * NS
LANES = _INFO.num_lanes

CHUNK = 128
PER_W = BATCH // NW
N_ITERS = PER_W // CHUNK
N_SPANS = CHUNK * NUM_FIELDS // LANES


def _fam_body(x_hbm, w_hbm, out_hbm, xb, eb, ob, sem):
    wid = lax.axis_index("s") * NC + lax.axis_index("c")
    iota = lax.iota(jnp.int32, LANES)

    def chunk_body(t, _):
        base = wid * PER_W + t * CHUNK

        pltpu.sync_copy(
            x_hbm.at[pl.ds(base * NUM_FIELDS, CHUNK * NUM_FIELDS)], xb)

        def span_body(sp, offv):
            v = xb[pl.ds(sp * LANES, LANES)]
            xb[pl.ds(sp * LANES, LANES)] = v + offv
            nxt = offv + LANES * VOCAB
            return jnp.where(nxt >= NUM_FIELDS * VOCAB,
                             nxt - NUM_FIELDS * VOCAB, nxt)

        lax.fori_loop(0, N_SPANS, span_body, iota * VOCAB)

        pltpu.async_copy(w_hbm.at[xb], eb, sem).wait()

        dspl = [jnp.full((LANES,), d, jnp.int32) for d in range(EMBED_DIM)]

        def tree_sum(vs):
            while len(vs) > 1:
                nxt = [vs[k] + vs[k + 1] for k in range(0, len(vs) - 1, 2)]
                if len(vs) % 2:
                    nxt.append(vs[-1])
                vs = nxt
            return vs[0]

        def group_body(g, _):
            rows = g * LANES + iota
            rb = rows * NUM_FIELDS
            rowsm = rows * NUM_PAIRS
            for i in range(NUM_FIELDS - 1):
                ei = [plsc.load_gather(eb, [rb + i, dspl[d]])
                      for d in range(EMBED_DIM)]
                pconst = i * (2 * NUM_FIELDS - i - 1) // 2 - i - 1

                def j_body(j, _, ei=ei, rb=rb, rowsm=rowsm, pconst=pconst):
                    rj = rb + j
                    prods = [ei[d] * plsc.load_gather(eb, [rj, dspl[d]])
                             for d in range(EMBED_DIM)]
                    acc = tree_sum(prods)
                    plsc.store_scatter(ob, [rowsm + (pconst + j)], acc)
                    return 0

                lax.fori_loop(i + 1, NUM_FIELDS, j_body, 0)
            return 0

        lax.fori_loop(0, CHUNK // LANES, group_body, 0)

        pltpu.sync_copy(ob, out_hbm.at[pl.ds(base * NUM_PAIRS,
                                             CHUNK * NUM_PAIRS)])
        return 0

    lax.fori_loop(0, N_ITERS, chunk_body, 0)


@jax.jit
def _fam(x_flat, w_flat):
    mesh = plsc.VectorSubcoreMesh(core_axis_name="c", subcore_axis_name="s")
    return pl.kernel(
        _fam_body,
        out_type=jax.ShapeDtypeStruct((BATCH * NUM_PAIRS,), jnp.float32),
        mesh=mesh,
        compiler_params=pltpu.CompilerParams(
            needs_layout_passes=False, use_tc_tiling_on_sc=False),
        scratch_types=[
            pltpu.VMEM((CHUNK * NUM_FIELDS,), jnp.int32),
            pltpu.VMEM((CHUNK * NUM_FIELDS, EMBED_DIM), jnp.float32),
            pltpu.VMEM((CHUNK * NUM_PAIRS,), jnp.float32),
            pltpu.SemaphoreType.DMA,
        ],
    )(x_flat, w_flat)


def kernel(x, W):
    x_flat = x.astype(jnp.int32).reshape(-1)
    w_flat = W.reshape(NUM_FIELDS * VOCAB, EMBED_DIM)
    return _fam(x_flat, w_flat).reshape(BATCH, NUM_PAIRS)


# speedup vs baseline: 1.7073x; 1.7073x over previous
import jax
import jax.numpy as jnp
from jax import lax
from jax.experimental import pallas as pl
from jax.experimental.pallas import tpu as pltpu
from jax.experimental.pallas import tpu_sc as plsc

NUM_FIELDS = 26
VOCAB = 100000
EMBED_DIM = 16
BATCH = 16384
NUM_PAIRS = (NUM_FIELDS * (NUM_FIELDS - 1)) // 2

_INFO = plsc.get_sparse_core_info()
NC = _INFO.num_cores
NS = _INFO.num_subcores
NW = NC * NS
LANES = _INFO.num_lanes

CHUNK = 64
PER_W = BATCH // NW
N_ITERS = PER_W // CHUNK
N_SPANS = CHUNK * NUM_FIELDS // LANES
PITCH = CHUNK + 1
FSTRIDE = EMBED_DIM * PITCH


def _fam_body(x_hbm, w_hbm, out_hbm, xb, eb2, ebt, ob, sem):
    wid = lax.axis_index("s") * NC + lax.axis_index("c")
    iota = lax.iota(jnp.int32, LANES)
    iota_pitch = iota * PITCH

    def chunk_body(t, _):
        base = wid * PER_W + t * CHUNK

        pltpu.sync_copy(
            x_hbm.at[pl.ds(base * NUM_FIELDS, CHUNK * NUM_FIELDS)], xb)

        def span_body(sp, offv):
            v = xb[pl.ds(sp * LANES, LANES)]
            xb[pl.ds(sp * LANES, LANES)] = v + offv
            nxt = offv + LANES * VOCAB
            return jnp.where(nxt >= NUM_FIELDS * VOCAB,
                             nxt - NUM_FIELDS * VOCAB, nxt)

        lax.fori_loop(0, N_SPANS, span_body, iota * VOCAB)

        pltpu.async_copy(w_hbm.at[xb], eb2, sem).wait()

        def tr_body(s, _):
            for f in range(NUM_FIELDS):
                v = eb2[s * NUM_FIELDS + f]
                tvec = iota_pitch + (f * FSTRIDE + s)
                plsc.store_scatter(ebt, [tvec], v)
            return 0

        lax.fori_loop(0, CHUNK, tr_body, 0)

        def group_body(g, _):
            g16 = g * LANES
            rowsm = (g16 + iota) * NUM_PAIRS
            for i in range(NUM_FIELDS - 1):
                ei = [ebt[pl.ds(g16 + (i * FSTRIDE + d * PITCH), LANES)]
                      for d in range(EMBED_DIM)]
                pconst = i * (2 * NUM_FIELDS - i - 1) // 2 - i - 1

                def j_body(j, _, ei=ei, g16=g16, rowsm=rowsm, pconst=pconst):
                    jbase = g16 + j * FSTRIDE
                    prods = [ei[d] * ebt[pl.ds(jbase + d * PITCH, LANES)]
                             for d in range(EMBED_DIM)]
                    while len(prods) > 1:
                        nxt = [prods[k] + prods[k + 1]
                               for k in range(0, len(prods) - 1, 2)]
                        if len(prods) % 2:
                            nxt.append(prods[-1])
                        prods = nxt
                    plsc.store_scatter(ob, [rowsm + (pconst + j)], prods[0])
                    return 0

                lax.fori_loop(i + 1, NUM_FIELDS, j_body, 0)
            return 0

        lax.fori_loop(0, CHUNK // LANES, group_body, 0)

        pltpu.sync_copy(ob, out_hbm.at[pl.ds(base * NUM_PAIRS,
                                             CHUNK * NUM_PAIRS)])
        return 0

    lax.fori_loop(0, N_ITERS, chunk_body, 0)


@jax.jit
def _fam(x_flat, w_flat):
    mesh = plsc.VectorSubcoreMesh(core_axis_name="c", subcore_axis_name="s")
    return pl.kernel(
        _fam_body,
        out_type=jax.ShapeDtypeStruct((BATCH * NUM_PAIRS,), jnp.float32),
        mesh=mesh,
        compiler_params=pltpu.CompilerParams(
            needs_layout_passes=False, use_tc_tiling_on_sc=False),
        scratch_types=[
            pltpu.VMEM((CHUNK * NUM_FIELDS,), jnp.int32),
            pltpu.VMEM((CHUNK * NUM_FIELDS, EMBED_DIM), jnp.float32),
            pltpu.VMEM((NUM_FIELDS * FSTRIDE,), jnp.float32),
            pltpu.VMEM((CHUNK * NUM_PAIRS,), jnp.float32),
            pltpu.SemaphoreType.DMA,
        ],
    )(x_flat, w_flat)


def kernel(x, W):
    x_flat = x.astype(jnp.int32).reshape(-1)
    w_flat = W.reshape(NUM_FIELDS * VOCAB, EMBED_DIM)
    return _fam(x_flat, w_flat).reshape(BATCH, NUM_PAIRS)


# speedup vs baseline: 1.7194x; 1.0071x over previous
import jax
import jax.numpy as jnp
from jax import lax
from jax.experimental import pallas as pl
from jax.experimental.pallas import tpu as pltpu
from jax.experimental.pallas import tpu_sc as plsc

NUM_FIELDS = 26
VOCAB = 100000
EMBED_DIM = 16
BATCH = 16384
NUM_PAIRS = (NUM_FIELDS * (NUM_FIELDS - 1)) // 2

_INFO = plsc.get_sparse_core_info()
NC = _INFO.num_cores
NS = _INFO.num_subcores
NW = NC * NS
LANES = _INFO.num_lanes

CHUNK = 64
PER_W = BATCH // NW
N_ITERS = PER_W // CHUNK
N_SPANS = CHUNK * NUM_FIELDS // LANES
PITCH = CHUNK + 1
FSTRIDE = EMBED_DIM * PITCH


def _fam_body(x_hbm, w_hbm, out_hbm, xb, eb2, ebt, ob, sem):
    wid = lax.axis_index("s") * NC + lax.axis_index("c")
    iota = lax.iota(jnp.int32, LANES)
    iota_pitch = iota * PITCH

    def chunk_body(t, _):
        base = wid * PER_W + t * CHUNK

        pltpu.sync_copy(
            x_hbm.at[pl.ds(base * NUM_FIELDS, CHUNK * NUM_FIELDS)], xb)

        def span_body(sp, offv):
            v = xb[pl.ds(sp * LANES, LANES)]
            xb[pl.ds(sp * LANES, LANES)] = v + offv
            nxt = offv + LANES * VOCAB
            return jnp.where(nxt >= NUM_FIELDS * VOCAB,
                             nxt - NUM_FIELDS * VOCAB, nxt)

        lax.fori_loop(0, N_SPANS, span_body, iota * VOCAB)

        pltpu.async_copy(w_hbm.at[xb], eb2, sem).wait()

        @plsc.parallel_loop(0, CHUNK, unroll=2)
        def tr_body(s):
            for f in range(NUM_FIELDS):
                v = eb2[s * NUM_FIELDS + f]
                tvec = iota_pitch + (f * FSTRIDE + s)
                plsc.store_scatter(ebt, [tvec], v)

        def group_body(g, _):
            g16 = g * LANES
            rowsm = (g16 + iota) * NUM_PAIRS
            for i in range(NUM_FIELDS - 1):
                ei = [ebt[pl.ds(g16 + (i * FSTRIDE + d * PITCH), LANES)]
                      for d in range(EMBED_DIM)]
                pconst = i * (2 * NUM_FIELDS - i - 1) // 2 - i - 1

                @plsc.parallel_loop(i + 1, NUM_FIELDS, unroll=4)
                def j_body(j, ei=ei, g16=g16, rowsm=rowsm, pconst=pconst):
                    jbase = g16 + j * FSTRIDE
                    prods = [ei[d] * ebt[pl.ds(jbase + d * PITCH, LANES)]
                             for d in range(EMBED_DIM)]
                    while len(prods) > 1:
                        nxt = [prods[k] + prods[k + 1]
                               for k in range(0, len(prods) - 1, 2)]
                        if len(prods) % 2:
                            nxt.append(prods[-1])
                        prods = nxt
                    plsc.store_scatter(ob, [rowsm + (pconst + j)], prods[0])
            return 0

        lax.fori_loop(0, CHUNK // LANES, group_body, 0)

        pltpu.sync_copy(ob, out_hbm.at[pl.ds(base * NUM_PAIRS,
                                             CHUNK * NUM_PAIRS)])
        return 0

    lax.fori_loop(0, N_ITERS, chunk_body, 0)


@jax.jit
def _fam(x_flat, w_flat):
    mesh = plsc.VectorSubcoreMesh(core_axis_name="c", subcore_axis_name="s")
    return pl.kernel(
        _fam_body,
        out_type=jax.ShapeDtypeStruct((BATCH * NUM_PAIRS,), jnp.float32),
        mesh=mesh,
        compiler_params=pltpu.CompilerParams(
            needs_layout_passes=False, use_tc_tiling_on_sc=False),
        scratch_types=[
            pltpu.VMEM((CHUNK * NUM_FIELDS,), jnp.int32),
            pltpu.VMEM((CHUNK * NUM_FIELDS, EMBED_DIM), jnp.float32),
            pltpu.VMEM((NUM_FIELDS * FSTRIDE,), jnp.float32),
            pltpu.VMEM((CHUNK * NUM_PAIRS,), jnp.float32),
            pltpu.SemaphoreType.DMA,
        ],
    )(x_flat, w_flat)


def kernel(x, W):
    x_flat = x.astype(jnp.int32).reshape(-1)
    w_flat = W.reshape(NUM_FIELDS * VOCAB, EMBED_DIM)
    return _fam(x_flat, w_flat).reshape(BATCH, NUM_PAIRS)
